# baseline (speedup 1.0000x reference)
import jax
import jax.numpy as jnp
from jax.experimental import pallas as pl
from jax.experimental.pallas import tpu as pltpu

_FEAT = 22
_LANES = 128


def _body(idx_ref, tbl_ref, out_ref):
    col = idx_ref[0] % _LANES
    x = jnp.transpose(tbl_ref[...])
    sub = jax.lax.broadcasted_iota(jnp.int32, (_LANES, _FEAT), 0)
    out_ref[...] = jnp.sum(
        jnp.where(sub == col, x, 0.0), axis=0, keepdims=True
    )


_GRID_SPEC = pltpu.PrefetchScalarGridSpec(
    num_scalar_prefetch=1,
    grid=(1,),
    in_specs=[
        pl.BlockSpec(
            (_FEAT, _LANES), lambda i, idx_ref: (0, idx_ref[0] // _LANES)
        ),
    ],
    out_specs=pl.BlockSpec((1, _FEAT), lambda i, idx_ref: (0, 0)),
)

_lookup = pl.pallas_call(
    _body,
    grid_spec=_GRID_SPEC,
    out_shape=jax.ShapeDtypeStruct((1, _FEAT), jnp.float32),
    compiler_params=pltpu.CompilerParams(
        skip_device_barrier=True,
        disable_bounds_checks=True,
        disable_semaphore_checks=True,
    ),
)


def kernel(index, table):
    idx = jnp.asarray(index, dtype=jnp.int32).reshape((1,))
    tbl_t = pltpu.with_memory_space_constraint(
        table.T, pltpu.MemorySpace.HBM
    )
    return _lookup(idx, tbl_t)
